# baseline (device time: 86074 ns/iter reference)
import jax
import jax.numpy as jnp
from jax import lax
from jax.experimental import pallas as pl
from jax.experimental.pallas import tpu as pltpu

W = 16
N = 2048
D = 512
H = 1024
E_LOC = 4
Q = H // 4
GROUP = N // 4
CHUNK = N // W


def kernel(x, router_W, route_idx, expert_W):
    del router_W

    def body(x_ref, idx_ref, w_ref, out_ref, xb_ref, wb_ref, work,
             sbuf, spos, rbuf, rpos, bufS,
             sp_s, sp_r, si_s, si_r, s2_s, s2_r, s3_s, s3_r,
             p4_s, p4_r, z4_s, z4_r):
        my = lax.axis_index("i")
        k = lax.rem(my, 4)
        z = lax.div(my, 4)
        p_right = 4 * z + lax.rem(k + 1, 4)
        p_left = 4 * z + lax.rem(k + 3, 4)
        z_right = 4 * lax.rem(z + 1, 4) + k
        z_left = 4 * lax.rem(z + 3, 4) + k

        def m4(v):
            return lax.rem(v + 8, 4)

        def mk(rows, nrows, c0, dst, ssem, rsem, d, s, dev):
            return pltpu.make_async_remote_copy(
                src_ref=work.at[pl.ds(rows, nrows), pl.ds(c0, Q)],
                dst_ref=dst,
                send_sem=ssem.at[d, s],
                recv_sem=rsem.at[d, s],
                device_id=(dev,),
                device_id_type=pl.DeviceIdType.MESH,
            )

        xb_ref[...] = x_ref[...].astype(jnp.bfloat16)
        wb_ref[...] = w_ref[...].astype(jnp.bfloat16)

        P = 64
        tril = jnp.tril(jnp.ones((GROUP, GROUP), jnp.float32))

        iota_p = lax.broadcasted_iota(
            jnp.int32, (GROUP, P), 1).astype(jnp.float32)

        def compute_compact(g, inst):
            c0 = inst * 2 * Q
            rows = pl.ds(g * GROUP, GROUP)
            xb = xb_ref[rows, :]
            ib = idx_ref[rows, :]
            e0 = my * E_LOC
            lf = (ib >= e0) & (ib < e0 + E_LOC)
            pos = jnp.dot(tril, lf.astype(jnp.float32),
                          preferred_element_type=jnp.float32) - 1.0
            OT = jnp.where(lf, (pos == iota_p).astype(jnp.bfloat16),
                           jnp.bfloat16(0.0))
            xg = lax.dot_general(OT, xb, (((0,), (0,)), ((), ())),
                                 preferred_element_type=jnp.float32
                                 ).astype(jnp.bfloat16)
            eg = lax.dot_general(OT, (ib - e0).astype(jnp.bfloat16),
                                 (((0,), (0,)), ((), ())),
                                 preferred_element_type=jnp.float32)
            acc = jnp.zeros((P, 2 * Q), jnp.float32)
            for e in range(E_LOC):
                m = eg == float(e)
                acc = acc + jnp.dot(
                    jnp.where(m, xg, jnp.bfloat16(0.0)),
                    wb_ref[e, :, c0:c0 + 2 * Q],
                    preferred_element_type=jnp.float32)
            return OT, acc, jnp.where(lf, pos, -1.0)

        pgf = m4(k + 1)
        pgb = m4(k + 3)
        zgf = m4(z + 1)
        zgb = m4(z + 3)

        p1_sends = []
        own = {}

        def p1_send(q, slot, src_cols, acc, posm, dev):
            sbuf[q, slot] = acc[:, src_cols:src_cols + Q].astype(jnp.bfloat16)
            spos[q, slot] = posm
            rp = pltpu.make_async_remote_copy(
                src_ref=sbuf.at[q, slot], dst_ref=rbuf.at[q, slot],
                send_sem=sp_s.at[q, slot], recv_sem=sp_r.at[q, slot],
                device_id=(dev,), device_id_type=pl.DeviceIdType.MESH)
            ri = pltpu.make_async_remote_copy(
                src_ref=spos.at[q, slot], dst_ref=rpos.at[q, slot],
                send_sem=si_s.at[q, slot], recv_sem=si_r.at[q, slot],
                device_id=(dev,), device_id_type=pl.DeviceIdType.MESH)
            rp.start()
            ri.start()
            p1_sends.append(rp)
            p1_sends.append(ri)

        for inst, g_off in ((0, 0), (1, 0), (0, 2), (1, 2),
                            (0, 1), (1, 1), (0, 3), (1, 3)):
            base = k if inst == 0 else z
            g = m4(base + g_off)
            OT, acc, posm = compute_compact(g, inst)
            if g_off == 1:
                own[(inst, 0)] = (OT, acc)
            else:
                o = m4(base + g_off - 1)
                dev = 4 * z + o if inst == 0 else 4 * o + k
                p1_send(inst * 2, m4(1 - g_off) - 1, 0, acc, posm, dev)
            if g_off == 3:
                own[(inst, 1)] = (OT, acc)
            else:
                o = m4(base + g_off + 1)
                dev = 4 * z + o if inst == 0 else 4 * o + k
                p1_send(inst * 2 + 1, m4(-1 - g_off) - 1, Q, acc, posm, dev)

        for inst in (0, 1):
            base = k if inst == 0 else z
            for dirn in (0, 1):
                q = inst * 2 + dirn
                g_own = m4(base + (1 if dirn == 0 else 3))
                OT, acc = own[(inst, dirn)]
                tot = jnp.dot(
                    OT, acc[:, dirn * Q:(dirn + 1) * Q].astype(jnp.bfloat16),
                    preferred_element_type=jnp.float32)
                for slot in range(3):
                    for sref, dsem, ssem in ((rbuf, sp_r, sp_s),
                                             (rpos, si_r, si_s)):
                        pltpu.make_async_remote_copy(
                            src_ref=sref.at[q, slot], dst_ref=sref.at[q, slot],
                            send_sem=ssem.at[q, slot],
                            recv_sem=dsem.at[q, slot],
                            device_id=(my,),
                            device_id_type=pl.DeviceIdType.MESH).wait_recv()
                    OTr = (rpos[q, slot] == iota_p).astype(jnp.bfloat16)
                    tot = tot + jnp.dot(OTr, rbuf[q, slot],
                                        preferred_element_type=jnp.float32)
                c0 = inst * 2 * Q + dirn * Q
                work[pl.ds(g_own * GROUP, GROUP), c0:c0 + Q] = (
                    tot.astype(jnp.bfloat16))

        p2_sends = []
        for d in (1, 2, 3):
            jz = m4(z + d)
            jk = m4(k + d)
            slot = 3 - d
            r = [mk(pgf * GROUP + jz * CHUNK, CHUNK, 0,
                    bufS.at[0, slot], s2_s, s2_r, 0, slot, 4 * jz + k),
                 mk(pgb * GROUP + jz * CHUNK, CHUNK, Q,
                    bufS.at[1, slot], s2_s, s2_r, 1, slot, 4 * jz + k),
                 mk(zgf * GROUP + jk * CHUNK, CHUNK, 2 * Q,
                    bufS.at[2, slot], s2_s, s2_r, 2, slot, 4 * z + jk),
                 mk(zgb * GROUP + jk * CHUNK, CHUNK, 3 * Q,
                    bufS.at[3, slot], s2_s, s2_r, 3, slot, 4 * z + jk)]
            for x_ in r:
                x_.start()
            p2_sends += r
        for x_ in p1_sends:
            x_.wait_send()
        for q in range(4):
            for slot in range(3):
                pltpu.make_async_remote_copy(
                    src_ref=bufS.at[q, slot], dst_ref=bufS.at[q, slot],
                    send_sem=s2_s.at[q, slot], recv_sem=s2_r.at[q, slot],
                    device_id=(my,), device_id_type=pl.DeviceIdType.MESH,
                ).wait_recv()
        rows = pl.ds(pgf * GROUP + z * CHUNK, CHUNK)
        work[rows, 0:Q] = work[rows, 0:Q] + (bufS[0, 0] + bufS[0, 1]
                                             + bufS[0, 2])
        rows = pl.ds(pgb * GROUP + z * CHUNK, CHUNK)
        work[rows, Q:2 * Q] = work[rows, Q:2 * Q] + (bufS[1, 0] + bufS[1, 1]
                                                     + bufS[1, 2])
        rows = pl.ds(zgf * GROUP + k * CHUNK, CHUNK)
        work[rows, 2 * Q:3 * Q] = work[rows, 2 * Q:3 * Q] + (
            bufS[2, 0] + bufS[2, 1] + bufS[2, 2])
        rows = pl.ds(zgb * GROUP + k * CHUNK, CHUNK)
        work[rows, 3 * Q:4 * Q] = work[rows, 3 * Q:4 * Q] + (
            bufS[3, 0] + bufS[3, 1] + bufS[3, 2])

        p3_sends = []
        rpf = pgf * GROUP + z * CHUNK
        rpb = pgb * GROUP + z * CHUNK
        rzf = zgf * GROUP + k * CHUNK
        rzb = zgb * GROUP + k * CHUNK
        for d in (1, 2, 3):
            jz = m4(z + d)
            jk = m4(k + d)
            slot = 3 - d
            r = [mk(rpf, CHUNK, 0,
                    work.at[pl.ds(rpf, CHUNK), pl.ds(0, Q)],
                    s3_s, s3_r, 0, slot, 4 * jz + k),
                 mk(rpb, CHUNK, Q,
                    work.at[pl.ds(rpb, CHUNK), pl.ds(Q, Q)],
                    s3_s, s3_r, 1, slot, 4 * jz + k),
                 mk(rzf, CHUNK, 2 * Q,
                    work.at[pl.ds(rzf, CHUNK), pl.ds(2 * Q, Q)],
                    s3_s, s3_r, 2, slot, 4 * z + jk),
                 mk(rzb, CHUNK, 3 * Q,
                    work.at[pl.ds(rzb, CHUNK), pl.ds(3 * Q, Q)],
                    s3_s, s3_r, 3, slot, 4 * z + jk)]
            for x_ in r:
                x_.start()
            p3_sends += r
        for x_ in p2_sends:
            x_.wait_send()
        for q in range(4):
            for slot in range(3):
                pltpu.make_async_remote_copy(
                    src_ref=bufS.at[q, slot], dst_ref=bufS.at[q, slot],
                    send_sem=s3_s.at[q, slot], recv_sem=s3_r.at[q, slot],
                    device_id=(my,), device_id_type=pl.DeviceIdType.MESH,
                ).wait_recv()

        def cast_quarters(gq0, gq1, gq2, gq3):
            out_ref[pl.ds(gq0 * GROUP, GROUP), 0:Q] = (
                work[pl.ds(gq0 * GROUP, GROUP), 0:Q].astype(jnp.float32))
            out_ref[pl.ds(gq1 * GROUP, GROUP), Q:2 * Q] = (
                work[pl.ds(gq1 * GROUP, GROUP), Q:2 * Q].astype(jnp.float32))
            out_ref[pl.ds(gq2 * GROUP, GROUP), 2 * Q:3 * Q] = (
                work[pl.ds(gq2 * GROUP, GROUP), 2 * Q:3 * Q].astype(
                    jnp.float32))
            out_ref[pl.ds(gq3 * GROUP, GROUP), 3 * Q:4 * Q] = (
                work[pl.ds(gq3 * GROUP, GROUP), 3 * Q:4 * Q].astype(
                    jnp.float32))

        for t in range(3):
            rpf = m4(k + 1 - t) * GROUP
            rpb = m4(k + 3 + t) * GROUP
            rzf = m4(z + 1 - t) * GROUP
            rzb = m4(z + 3 + t) * GROUP
            r = [mk(rpf, GROUP, 0,
                    work.at[pl.ds(rpf, GROUP), pl.ds(0, Q)],
                    p4_s, p4_r, 0, t, p_right),
                 mk(rpb, GROUP, Q,
                    work.at[pl.ds(rpb, GROUP), pl.ds(Q, Q)],
                    p4_s, p4_r, 1, t, p_left),
                 mk(rzf, GROUP, 2 * Q,
                    work.at[pl.ds(rzf, GROUP), pl.ds(2 * Q, Q)],
                    z4_s, z4_r, 0, t, z_right),
                 mk(rzb, GROUP, 3 * Q,
                    work.at[pl.ds(rzb, GROUP), pl.ds(3 * Q, Q)],
                    z4_s, z4_r, 1, t, z_left)]
            for x_ in r:
                x_.start()
            if t == 0:
                for x_ in p3_sends:
                    x_.wait_send()
                cast_quarters(pgf, pgb, zgf, zgb)
            else:
                cast_quarters(m4(k - t + 1), m4(k + t - 1),
                              m4(z - t + 1), m4(z + t - 1))
            for x_ in r:
                x_.wait()
        cast_quarters(m4(k - 2), m4(k + 2), m4(z - 2), m4(z + 2))

    bf16 = jnp.bfloat16
    dma23 = pltpu.SemaphoreType.DMA((2, 3))
    return pl.pallas_call(
        body,
        out_shape=jax.ShapeDtypeStruct((N, H), jnp.float32),
        in_specs=[
            pl.BlockSpec(memory_space=pltpu.VMEM),
            pl.BlockSpec(memory_space=pltpu.VMEM),
            pl.BlockSpec(memory_space=pltpu.VMEM),
        ],
        out_specs=pl.BlockSpec(memory_space=pltpu.VMEM),
        scratch_shapes=[
            pltpu.VMEM((N, D), bf16),
            pltpu.VMEM((E_LOC, D, H), bf16),
            pltpu.VMEM((N, H), bf16),
            pltpu.VMEM((4, 3, 64, Q), bf16),
            pltpu.VMEM((4, 3, GROUP, 1), jnp.float32),
            pltpu.VMEM((4, 3, 64, Q), bf16),
            pltpu.VMEM((4, 3, GROUP, 1), jnp.float32),
            pltpu.VMEM((4, 3, CHUNK, Q), bf16),
            pltpu.SemaphoreType.DMA((4, 3)),
            pltpu.SemaphoreType.DMA((4, 3)),
            pltpu.SemaphoreType.DMA((4, 3)),
            pltpu.SemaphoreType.DMA((4, 3)),
            pltpu.SemaphoreType.DMA((4, 3)),
            pltpu.SemaphoreType.DMA((4, 3)),
            pltpu.SemaphoreType.DMA((4, 3)),
            pltpu.SemaphoreType.DMA((4, 3)),
            dma23, dma23, dma23, dma23,
        ],
    )(x, route_idx, expert_W)


# device time: 81834 ns/iter; 1.0518x vs baseline; 1.0518x over previous
import jax
import jax.numpy as jnp
from jax import lax
from jax.experimental import pallas as pl
from jax.experimental.pallas import tpu as pltpu

W = 16
N = 2048
D = 512
H = 1024
E_LOC = 4
Q = H // 4
GROUP = N // 4
CHUNK = N // W


def kernel(x, router_W, route_idx, expert_W):
    del router_W

    def body(x_ref, idx_ref, w_ref, out_ref, xb_ref, wb_ref, work,
             bufP1, bufZ1, bufS,
             p1_s, p1_r, z1_s, z1_r, s2_s, s2_r, s3_s, s3_r,
             p4_s, p4_r, z4_s, z4_r):
        my = lax.axis_index("i")
        k = lax.rem(my, 4)
        z = lax.div(my, 4)
        p_right = 4 * z + lax.rem(k + 1, 4)
        p_left = 4 * z + lax.rem(k + 3, 4)
        z_right = 4 * lax.rem(z + 1, 4) + k
        z_left = 4 * lax.rem(z + 3, 4) + k

        def m4(v):
            return lax.rem(v + 8, 4)

        def mk(rows, nrows, c0, dst, ssem, rsem, d, s, dev):
            return pltpu.make_async_remote_copy(
                src_ref=work.at[pl.ds(rows, nrows), pl.ds(c0, Q)],
                dst_ref=dst,
                send_sem=ssem.at[d, s],
                recv_sem=rsem.at[d, s],
                device_id=(dev,),
                device_id_type=pl.DeviceIdType.MESH,
            )

        xb_ref[...] = x_ref[...].astype(jnp.bfloat16)
        wb_ref[...] = w_ref[...].astype(jnp.bfloat16)

        P = 64
        tril = jnp.tril(jnp.ones((GROUP, GROUP), jnp.float32))

        def compute_half(g, half):
            c0 = half * 2 * Q
            rows = pl.ds(g * GROUP, GROUP)
            xb = xb_ref[rows, :]
            ib = idx_ref[rows, :]
            e0 = my * E_LOC
            lf = (ib >= e0) & (ib < e0 + E_LOC)
            lf32 = lf.astype(jnp.float32)
            pos = jnp.dot(tril, lf32,
                          preferred_element_type=jnp.float32) - 1.0
            iota_p = lax.broadcasted_iota(
                jnp.int32, (GROUP, P), 1).astype(jnp.float32)
            OT = jnp.where(lf, (pos == iota_p).astype(jnp.bfloat16),
                           jnp.bfloat16(0.0))
            xg = lax.dot_general(OT, xb, (((0,), (0,)), ((), ())),
                                 preferred_element_type=jnp.float32
                                 ).astype(jnp.bfloat16)
            eg = lax.dot_general(OT, (ib - e0).astype(jnp.bfloat16),
                                 (((0,), (0,)), ((), ())),
                                 preferred_element_type=jnp.float32)
            acc = jnp.zeros((P, 2 * Q), jnp.float32)
            for e in range(E_LOC):
                m = eg == float(e)
                acc = acc + jnp.dot(
                    jnp.where(m, xg, jnp.bfloat16(0.0)),
                    wb_ref[e, :, c0:c0 + 2 * Q],
                    preferred_element_type=jnp.float32)
            work[rows, c0:c0 + 2 * Q] = jnp.dot(
                OT, acc.astype(jnp.bfloat16),
                preferred_element_type=jnp.float32).astype(jnp.bfloat16)

        def phase1_start(s):
            r = [mk(m4(k - s) * GROUP, GROUP, 0, bufP1.at[0, s],
                    p1_s, p1_r, 0, s, p_right),
                 mk(m4(k + s) * GROUP, GROUP, Q, bufP1.at[1, s],
                    p1_s, p1_r, 1, s, p_left),
                 mk(m4(z - s) * GROUP, GROUP, 2 * Q, bufZ1.at[0, s],
                    z1_s, z1_r, 0, s, z_right),
                 mk(m4(z + s) * GROUP, GROUP, 3 * Q, bufZ1.at[1, s],
                    z1_s, z1_r, 1, s, z_left)]
            for x_ in r:
                x_.start()
            return r

        def phase1_finish(r, s):
            for x_ in r:
                x_.wait()
            rows = pl.ds(m4(k - s - 1) * GROUP, GROUP)
            work[rows, 0:Q] = work[rows, 0:Q] + bufP1[0, s]
            rows = pl.ds(m4(k + s + 1) * GROUP, GROUP)
            work[rows, Q:2 * Q] = work[rows, Q:2 * Q] + bufP1[1, s]
            rows = pl.ds(m4(z - s - 1) * GROUP, GROUP)
            work[rows, 2 * Q:3 * Q] = work[rows, 2 * Q:3 * Q] + bufZ1[0, s]
            rows = pl.ds(m4(z + s + 1) * GROUP, GROUP)
            work[rows, 3 * Q:4 * Q] = work[rows, 3 * Q:4 * Q] + bufZ1[1, s]

        compute_half(k, 0)
        compute_half(z, 1)
        r0 = phase1_start(0)
        compute_half(m4(k + 1), 0)
        compute_half(m4(k + 3), 0)
        compute_half(m4(z + 1), 1)
        compute_half(m4(z + 3), 1)
        phase1_finish(r0, 0)
        r1 = phase1_start(1)
        compute_half(m4(k + 2), 0)
        compute_half(m4(z + 2), 1)
        phase1_finish(r1, 1)
        r2 = phase1_start(2)
        phase1_finish(r2, 2)

        pgf = m4(k + 1)
        pgb = m4(k + 3)
        zgf = m4(z + 1)
        zgb = m4(z + 3)

        p2_sends = []
        for d in (1, 2, 3):
            jz = m4(z + d)
            jk = m4(k + d)
            slot = 3 - d
            r = [mk(pgf * GROUP + jz * CHUNK, CHUNK, 0,
                    bufS.at[0, slot], s2_s, s2_r, 0, slot, 4 * jz + k),
                 mk(pgb * GROUP + jz * CHUNK, CHUNK, Q,
                    bufS.at[1, slot], s2_s, s2_r, 1, slot, 4 * jz + k),
                 mk(zgf * GROUP + jk * CHUNK, CHUNK, 2 * Q,
                    bufS.at[2, slot], s2_s, s2_r, 2, slot, 4 * z + jk),
                 mk(zgb * GROUP + jk * CHUNK, CHUNK, 3 * Q,
                    bufS.at[3, slot], s2_s, s2_r, 3, slot, 4 * z + jk)]
            for x_ in r:
                x_.start()
            p2_sends += r
        for q in range(4):
            for slot in range(3):
                pltpu.make_async_remote_copy(
                    src_ref=bufS.at[q, slot], dst_ref=bufS.at[q, slot],
                    send_sem=s2_s.at[q, slot], recv_sem=s2_r.at[q, slot],
                    device_id=(my,), device_id_type=pl.DeviceIdType.MESH,
                ).wait_recv()
        rows = pl.ds(pgf * GROUP + z * CHUNK, CHUNK)
        work[rows, 0:Q] = work[rows, 0:Q] + (bufS[0, 0] + bufS[0, 1]
                                             + bufS[0, 2])
        rows = pl.ds(pgb * GROUP + z * CHUNK, CHUNK)
        work[rows, Q:2 * Q] = work[rows, Q:2 * Q] + (bufS[1, 0] + bufS[1, 1]
                                                     + bufS[1, 2])
        rows = pl.ds(zgf * GROUP + k * CHUNK, CHUNK)
        work[rows, 2 * Q:3 * Q] = work[rows, 2 * Q:3 * Q] + (
            bufS[2, 0] + bufS[2, 1] + bufS[2, 2])
        rows = pl.ds(zgb * GROUP + k * CHUNK, CHUNK)
        work[rows, 3 * Q:4 * Q] = work[rows, 3 * Q:4 * Q] + (
            bufS[3, 0] + bufS[3, 1] + bufS[3, 2])

        p3_sends = []
        rpf = pgf * GROUP + z * CHUNK
        rpb = pgb * GROUP + z * CHUNK
        rzf = zgf * GROUP + k * CHUNK
        rzb = zgb * GROUP + k * CHUNK
        for d in (1, 2, 3):
            jz = m4(z + d)
            jk = m4(k + d)
            slot = 3 - d
            r = [mk(rpf, CHUNK, 0,
                    work.at[pl.ds(rpf, CHUNK), pl.ds(0, Q)],
                    s3_s, s3_r, 0, slot, 4 * jz + k),
                 mk(rpb, CHUNK, Q,
                    work.at[pl.ds(rpb, CHUNK), pl.ds(Q, Q)],
                    s3_s, s3_r, 1, slot, 4 * jz + k),
                 mk(rzf, CHUNK, 2 * Q,
                    work.at[pl.ds(rzf, CHUNK), pl.ds(2 * Q, Q)],
                    s3_s, s3_r, 2, slot, 4 * z + jk),
                 mk(rzb, CHUNK, 3 * Q,
                    work.at[pl.ds(rzb, CHUNK), pl.ds(3 * Q, Q)],
                    s3_s, s3_r, 3, slot, 4 * z + jk)]
            for x_ in r:
                x_.start()
            p3_sends += r
        for x_ in p2_sends:
            x_.wait_send()
        for q in range(4):
            for slot in range(3):
                pltpu.make_async_remote_copy(
                    src_ref=bufS.at[q, slot], dst_ref=bufS.at[q, slot],
                    send_sem=s3_s.at[q, slot], recv_sem=s3_r.at[q, slot],
                    device_id=(my,), device_id_type=pl.DeviceIdType.MESH,
                ).wait_recv()

        def cast_quarters(gq0, gq1, gq2, gq3):
            out_ref[pl.ds(gq0 * GROUP, GROUP), 0:Q] = (
                work[pl.ds(gq0 * GROUP, GROUP), 0:Q].astype(jnp.float32))
            out_ref[pl.ds(gq1 * GROUP, GROUP), Q:2 * Q] = (
                work[pl.ds(gq1 * GROUP, GROUP), Q:2 * Q].astype(jnp.float32))
            out_ref[pl.ds(gq2 * GROUP, GROUP), 2 * Q:3 * Q] = (
                work[pl.ds(gq2 * GROUP, GROUP), 2 * Q:3 * Q].astype(
                    jnp.float32))
            out_ref[pl.ds(gq3 * GROUP, GROUP), 3 * Q:4 * Q] = (
                work[pl.ds(gq3 * GROUP, GROUP), 3 * Q:4 * Q].astype(
                    jnp.float32))

        for t in range(3):
            rpf = m4(k + 1 - t) * GROUP
            rpb = m4(k + 3 + t) * GROUP
            rzf = m4(z + 1 - t) * GROUP
            rzb = m4(z + 3 + t) * GROUP
            r = [mk(rpf, GROUP, 0,
                    work.at[pl.ds(rpf, GROUP), pl.ds(0, Q)],
                    p4_s, p4_r, 0, t, p_right),
                 mk(rpb, GROUP, Q,
                    work.at[pl.ds(rpb, GROUP), pl.ds(Q, Q)],
                    p4_s, p4_r, 1, t, p_left),
                 mk(rzf, GROUP, 2 * Q,
                    work.at[pl.ds(rzf, GROUP), pl.ds(2 * Q, Q)],
                    z4_s, z4_r, 0, t, z_right),
                 mk(rzb, GROUP, 3 * Q,
                    work.at[pl.ds(rzb, GROUP), pl.ds(3 * Q, Q)],
                    z4_s, z4_r, 1, t, z_left)]
            for x_ in r:
                x_.start()
            if t == 0:
                for x_ in p3_sends:
                    x_.wait_send()
                cast_quarters(pgf, pgb, zgf, zgb)
            else:
                cast_quarters(m4(k - t + 1), m4(k + t - 1),
                              m4(z - t + 1), m4(z + t - 1))
            for x_ in r:
                x_.wait()
        cast_quarters(m4(k - 2), m4(k + 2), m4(z - 2), m4(z + 2))

    bf16 = jnp.bfloat16
    dma23 = pltpu.SemaphoreType.DMA((2, 3))
    return pl.pallas_call(
        body,
        out_shape=jax.ShapeDtypeStruct((N, H), jnp.float32),
        in_specs=[
            pl.BlockSpec(memory_space=pltpu.VMEM),
            pl.BlockSpec(memory_space=pltpu.VMEM),
            pl.BlockSpec(memory_space=pltpu.VMEM),
        ],
        out_specs=pl.BlockSpec(memory_space=pltpu.VMEM),
        scratch_shapes=[
            pltpu.VMEM((N, D), bf16),
            pltpu.VMEM((E_LOC, D, H), bf16),
            pltpu.VMEM((N, H), bf16),
            pltpu.VMEM((2, 3, GROUP, Q), bf16),
            pltpu.VMEM((2, 3, GROUP, Q), bf16),
            pltpu.VMEM((4, 3, CHUNK, Q), bf16),
            dma23, dma23, dma23, dma23,
            pltpu.SemaphoreType.DMA((4, 3)),
            pltpu.SemaphoreType.DMA((4, 3)),
            pltpu.SemaphoreType.DMA((4, 3)),
            pltpu.SemaphoreType.DMA((4, 3)),
            dma23, dma23, dma23, dma23,
        ],
    )(x, route_idx, expert_W)
